# baseline (speedup 1.0000x reference)
import functools

import jax
import jax.numpy as jnp
from jax import lax
from jax.experimental import pallas as pl
from jax.experimental.pallas import tpu as pltpu
from jax.experimental.pallas import tpu_sc as plsc

NC = 2
NS = 16
NW = NC * NS
LANES = 16

CHUNK_ROWS = 32
PC = 800
SHARD = 64000


def _make_sc_kernel(B, L, V, E):
    assert E == LANES
    rows_per_w = B // NW
    nchunks = rows_per_w // CHUNK_ROWS
    ci = CHUNK_ROWS * L
    idx_per_w = rows_per_w * L
    inv_l = 1.0 / float(L)
    last_shard = V - SHARD * (NS - 1)
    assert 0 < last_shard <= SHARD and last_shard % PC == 0
    assert (SHARD // PC) % 2 == 0 and (last_shard // PC) % 2 == 0

    mesh = plsc.VectorSubcoreMesh(core_axis_name="c", subcore_axis_name="s")

    @functools.partial(
        pl.kernel,
        mesh=mesh,
        out_type=jax.ShapeDtypeStruct((B,), jnp.float32),
        compiler_params=pltpu.CompilerParams(
            needs_layout_passes=False, use_tc_tiling_on_sc=False),
        scratch_types=[
            pltpu.VMEM_SHARED((V,), jnp.float32),
            pltpu.VMEM((E, PC), jnp.float32),
            pltpu.VMEM((E, PC), jnp.float32),
            pltpu.VMEM((PC,), jnp.float32),
            pltpu.VMEM((ci,), jnp.int32),
            pltpu.VMEM((ci,), jnp.int32),
            pltpu.VMEM((ci,), jnp.float32),
            pltpu.VMEM((ci,), jnp.float32),
            pltpu.VMEM((rows_per_w,), jnp.float32),
            pltpu.VMEM((LANES,), jnp.float32),
            pltpu.VMEM((LANES,), jnp.float32),
            pltpu.SemaphoreType.DMA,
            pltpu.SemaphoreType.DMA,
            pltpu.SemaphoreType.DMA,
            pltpu.SemaphoreType.DMA,
        ],
    )
    def sc_kernel(x_hbm, embt_hbm, w_hbm, b_hbm, out_hbm,
                  p_sh, pin0, pin1, pout, idx0, idx1, vals0, vals1,
                  outbuf, wv, bv, psem0, psem1, sem0, sem1):
        cid = lax.axis_index("c")
        sid = lax.axis_index("s")
        wid = sid * NC + cid
        pltpu.sync_copy(w_hbm, wv)
        pltpu.sync_copy(b_hbm, bv)
        iota = lax.broadcasted_iota(jnp.int32, (LANES,), 0)
        wl = wv[...] * inv_l
        bvec = bv[...]

        shard_off = sid * SHARD
        shard_len = jnp.where(sid == NS - 1, last_shard, SHARD)
        nch = shard_len // PC
        pins = (pin0, pin1)
        psems = (psem0, psem1)

        def pstart(cc, k):
            off = shard_off + cc * PC
            return pltpu.async_copy(
                embt_hbm.at[:, pl.ds(off, PC)], pins[k], psems[k])

        def pchunk(cc, k):
            pin = pins[k]

            def qbody(q, carry):
                col = pl.ds(q * LANES, LANES)
                acc = pin[0, col] * wl[0]
                for e in range(1, E):
                    acc = acc + pin[e, col] * wl[e]
                pout[col] = acc
                return carry

            lax.fori_loop(0, PC // LANES, qbody, 0)
            off = shard_off + cc * PC
            pltpu.sync_copy(pout, p_sh.at[pl.ds(off, PC)])

        pstart(0, 0)

        def pbody2(t, carry):
            for k in (0, 1):
                cc = 2 * t + k
                cnext = jnp.minimum(cc + 1, nch - 1)
                pltpu.make_async_copy(
                    embt_hbm.at[:, pl.ds(shard_off, PC)],
                    pins[k], psems[k]).wait()
                pltpu.async_copy(
                    embt_hbm.at[:, pl.ds(shard_off + cnext * PC, PC)],
                    pins[1 - k], psems[1 - k])
                pchunk(cc, k)
            return carry

        lax.fori_loop(0, nch // 2, pbody2, 0)
        pltpu.make_async_copy(
            embt_hbm.at[:, pl.ds(shard_off, PC)], pins[0], psems[0]).wait()
        plsc.subcore_barrier()

        base_i = wid * idx_per_w
        idx_bufs = (idx0, idx1)
        val_bufs = (vals0, vals1)
        sems = (sem0, sem1)

        def gstart(c, k):
            off = base_i + c * ci
            pltpu.sync_copy(x_hbm.at[pl.ds(off, ci)], idx_bufs[k])
            return pltpu.async_copy(p_sh.at[idx_bufs[k]], val_bufs[k], sems[k])

        def reduce_chunk(c, k):
            vals_v = val_bufs[k]
            for g in range(CHUNK_ROWS // LANES):
                rowbase = (iota + g * LANES) * L

                def lbody(t, a):
                    l = t * 8
                    for u in range(8):
                        a = a + plsc.load_gather(vals_v, [rowbase + (l + u)])
                    return a

                acc = lax.fori_loop(0, L // 8, lbody,
                                    jnp.zeros((LANES,), jnp.float32))
                z = acc + bvec
                e = jnp.exp(-jnp.abs(z))
                s = jnp.where(z >= 0, 1.0 / (1.0 + e), e / (1.0 + e))
                outbuf[pl.ds(c * CHUNK_ROWS + g * LANES, LANES)] = s

        cpg = gstart(0, 0)
        for c in range(nchunks):
            cpg.wait()
            if c + 1 < nchunks:
                cpg = gstart(c + 1, (c + 1) % 2)
            reduce_chunk(c, c % 2)

        pltpu.sync_copy(outbuf, out_hbm.at[pl.ds(wid * rows_per_w, rows_per_w)])

    return sc_kernel


@jax.jit
def kernel(x, emb_table, lin_w, lin_b):
    B, L = x.shape
    V, E = emb_table.shape
    x_flat = x.reshape(-1).astype(jnp.int32)
    embt = emb_table.T
    w16 = lin_w.reshape(E).astype(jnp.float32)
    b16 = jnp.broadcast_to(lin_b.reshape(1), (LANES,)).astype(jnp.float32)
    out = _make_sc_kernel(B, L, V, E)(x_flat, embt, w16, b16)
    return out.reshape(B, 1)
